# baseline (speedup 1.0000x reference)
import functools

import jax
import jax.numpy as jnp
from jax import lax
from jax.experimental import pallas as pl
from jax.experimental.pallas import tpu as pltpu
from jax.experimental.pallas import tpu_sc as plsc

N = 320000
D = 128
NUM_SEG = 10000
NT = 16
HCHUNK = N // NT
OCHUNK = N // 32
SEG_PAD = 10240
SEG_PT = SEG_PAD // NT
L = 16

BN = 16000
G = N // BN


def _matvec_body(f_ref, w_ref, o_ref):
    f = f_ref[0]
    w = w_ref[...]
    o_ref[0] = jnp.exp(jax.lax.dot_general(
        w, f, (((1,), (1,)), ((), ())),
        preferred_element_type=jnp.float32))


def _matvec(features, gamma_w):
    f3 = features.reshape(G, BN, D)
    return pl.pallas_call(
        _matvec_body,
        grid=(G,),
        in_specs=[
            pl.BlockSpec((1, BN, D), lambda i: (i, 0, 0)),
            pl.BlockSpec((1, D), lambda i: (0, 0)),
        ],
        out_specs=pl.BlockSpec((1, 1, BN), lambda i: (i, 0, 0)),
        out_shape=jax.ShapeDtypeStruct((G, 1, BN), jnp.float32),
    )(f3, gamma_w).reshape(N)


_mesh = plsc.VectorSubcoreMesh(core_axis_name="c", subcore_axis_name="s")
_sc_params = pltpu.CompilerParams(needs_layout_passes=False,
                                  use_tc_tiling_on_sc=False)


@functools.partial(
    pl.kernel,
    out_type=jax.ShapeDtypeStruct((N,), jnp.float32),
    mesh=_mesh,
    compiler_params=_sc_params,
    scratch_types=[
        pltpu.VMEM((HCHUNK + L,), jnp.int32),
        pltpu.VMEM((HCHUNK,), jnp.float32),
        pltpu.VMEM((SEG_PAD,), jnp.float32),
        pltpu.VMEM((NT, SEG_PT), jnp.float32),
        pltpu.VMEM((OCHUNK,), jnp.float32),
        pltpu.VMEM_SHARED((NT, SEG_PAD), jnp.float32),
        pltpu.VMEM_SHARED((SEG_PAD,), jnp.float32),
    ],
)
def _segment(pid_hbm, logit_hbm, out_hbm, pid_v, logit_v, box_v, red_v, out_v,
             shared, shared_box):
    cid = lax.axis_index("c")
    sid = lax.axis_index("s")
    base = sid * HCHUNK
    pltpu.sync_copy(pid_hbm.at[pl.ds(base, HCHUNK)], pid_v.at[pl.ds(0, HCHUNK)])
    pltpu.sync_copy(logit_hbm.at[pl.ds(base, HCHUNK)], logit_v)

    def zero_body(i, _):
        box_v[pl.ds(i * L, L)] = jnp.zeros((L,), jnp.float32)
        return 0

    lax.fori_loop(0, SEG_PAD // L, zero_body, 0)

    lane = lax.iota(jnp.int32, L)
    m_last = lane == (L - 1)
    m_not_last = lane < (L - 1)

    def body(i, _):
        ids = pid_v[pl.ds(i * L, L)]
        ids_n = pid_v[pl.ds(i * L + 1, L)]
        vals = logit_v[pl.ds(i * L, L)]
        c = plsc.cumsum(vals)
        chg = ids != ids_n
        m_end = jnp.logical_or(chg, m_last)
        m_sub = jnp.logical_and(chg, m_not_last)
        plsc.addupdate_scatter(box_v, [ids], c, mask=m_end)
        plsc.addupdate_scatter(box_v, [ids_n], -c, mask=m_sub)
        return 0

    lax.fori_loop(0, HCHUNK // L, body, 0)

    pltpu.sync_copy(box_v, shared.at[sid])
    plsc.subcore_barrier()
    pltpu.sync_copy(shared.at[:, pl.ds(sid * SEG_PT, SEG_PT)], red_v)

    def red_body(j, _):
        def rbody(r, acc):
            return acc + red_v[r, pl.ds(j * L, L)]

        acc = lax.fori_loop(0, NT, rbody, jnp.zeros((L,), jnp.float32))
        box_v[pl.ds(j * L, L)] = acc
        return 0

    lax.fori_loop(0, SEG_PT // L, red_body, 0)

    pltpu.sync_copy(box_v.at[pl.ds(0, SEG_PT)],
                    shared_box.at[pl.ds(sid * SEG_PT, SEG_PT)])
    plsc.subcore_barrier()
    pltpu.sync_copy(shared_box, box_v)

    off = cid * OCHUNK

    def norm_body(i, _):
        ids = pid_v[pl.ds(off + i * L, L)]
        vals = logit_v[pl.ds(off + i * L, L)]
        part = plsc.load_gather(box_v, [ids])
        out_v[pl.ds(i * L, L)] = vals / part
        return 0

    lax.fori_loop(0, OCHUNK // L, norm_body, 0)
    pltpu.sync_copy(out_v, out_hbm.at[pl.ds(base + off, OCHUNK)])


def kernel(features, phrase_id, unique_phrase, gamma_w):
    logit = _matvec(features, gamma_w)
    weights = _segment(phrase_id, logit)
    return weights[:, None]


# speedup vs baseline: 1.2495x; 1.2495x over previous
import functools

import jax
import jax.numpy as jnp
from jax import lax
from jax.experimental import pallas as pl
from jax.experimental.pallas import tpu as pltpu
from jax.experimental.pallas import tpu_sc as plsc

N = 320000
D = 128
NUM_SEG = 10000
NT = 16
HCHUNK = N // NT
OCHUNK = N // 32
SEG_PAD = 10240
SEG_PT = SEG_PAD // NT
L = 16

BN = 16000
G = N // BN


def _matvec_body(f_ref, w_ref, o_ref):
    f = f_ref[0]
    w = w_ref[...]
    o_ref[0] = jnp.exp(jax.lax.dot_general(
        w, f, (((1,), (1,)), ((), ())),
        preferred_element_type=jnp.float32))


def _matvec(features, gamma_w):
    f3 = features.reshape(G, BN, D)
    return pl.pallas_call(
        _matvec_body,
        grid=(G,),
        in_specs=[
            pl.BlockSpec((1, BN, D), lambda i: (i, 0, 0)),
            pl.BlockSpec((1, D), lambda i: (0, 0)),
        ],
        out_specs=pl.BlockSpec((1, 1, BN), lambda i: (i, 0, 0)),
        out_shape=jax.ShapeDtypeStruct((G, 1, BN), jnp.float32),
    )(f3, gamma_w).reshape(N)


_mesh = plsc.VectorSubcoreMesh(core_axis_name="c", subcore_axis_name="s")
_sc_params = pltpu.CompilerParams(needs_layout_passes=False,
                                  use_tc_tiling_on_sc=False)


@functools.partial(
    pl.kernel,
    out_type=jax.ShapeDtypeStruct((N,), jnp.float32),
    mesh=_mesh,
    compiler_params=_sc_params,
    scratch_types=[
        pltpu.VMEM((HCHUNK + L,), jnp.int32),
        pltpu.VMEM((HCHUNK,), jnp.float32),
        pltpu.VMEM((SEG_PAD,), jnp.float32),
        pltpu.VMEM((NT, SEG_PT), jnp.float32),
        pltpu.VMEM((OCHUNK,), jnp.float32),
        pltpu.VMEM_SHARED((NT, SEG_PAD), jnp.float32),
        pltpu.VMEM_SHARED((SEG_PAD,), jnp.float32),
        pltpu.SemaphoreType.DMA,
        pltpu.SemaphoreType.DMA,
    ],
)
def _segment(pid_hbm, logit_hbm, out_hbm, pid_v, logit_v, box_v, red_v, out_v,
             shared, shared_box, sem_a, sem_b):
    cid = lax.axis_index("c")
    sid = lax.axis_index("s")
    base = sid * HCHUNK
    cp_a = pltpu.async_copy(pid_hbm.at[pl.ds(base, HCHUNK)],
                            pid_v.at[pl.ds(0, HCHUNK)], sem_a)
    cp_b = pltpu.async_copy(logit_hbm.at[pl.ds(base, HCHUNK)], logit_v, sem_b)

    @plsc.parallel_loop(0, SEG_PAD // L, unroll=4)
    def _zero(i):
        box_v[pl.ds(i * L, L)] = jnp.zeros((L,), jnp.float32)

    cp_a.wait()
    cp_b.wait()

    lane = lax.iota(jnp.int32, L)
    m_last = lane == (L - 1)
    m_not_last = lane < (L - 1)

    @plsc.parallel_loop(0, HCHUNK // L, unroll=4)
    def _body(i):
        k = i * L
        ids = pid_v[pl.ds(k, L)]
        ids_n = pid_v[pl.ds(k + 1, L)]
        vals = logit_v[pl.ds(k, L)]
        c = plsc.cumsum(vals)
        chg = ids != ids_n
        m_end = jnp.logical_or(chg, m_last)
        m_sub = jnp.logical_and(chg, m_not_last)
        plsc.addupdate_scatter(box_v, [ids], c, mask=m_end)
        plsc.addupdate_scatter(box_v, [ids_n], -c, mask=m_sub)

    pltpu.sync_copy(box_v, shared.at[sid])
    plsc.subcore_barrier()
    pltpu.sync_copy(shared.at[:, pl.ds(sid * SEG_PT, SEG_PT)], red_v)

    def red_body(j, _):
        def rbody(r, acc):
            return acc + red_v[r, pl.ds(j * L, L)]

        acc = lax.fori_loop(0, NT, rbody, jnp.zeros((L,), jnp.float32))
        box_v[pl.ds(j * L, L)] = acc
        return 0

    lax.fori_loop(0, SEG_PT // L, red_body, 0)

    pltpu.sync_copy(box_v.at[pl.ds(0, SEG_PT)],
                    shared_box.at[pl.ds(sid * SEG_PT, SEG_PT)])
    plsc.subcore_barrier()
    pltpu.sync_copy(shared_box, box_v)

    off = cid * OCHUNK

    @plsc.parallel_loop(0, OCHUNK // L, unroll=4)
    def _norm(i):
        k = i * L
        ids = pid_v[pl.ds(off + k, L)]
        vals = logit_v[pl.ds(off + k, L)]
        part = plsc.load_gather(box_v, [ids])
        out_v[pl.ds(k, L)] = vals / part
    pltpu.sync_copy(out_v, out_hbm.at[pl.ds(base + off, OCHUNK)])


def kernel(features, phrase_id, unique_phrase, gamma_w):
    logit = _matvec(features, gamma_w)
    weights = _segment(phrase_id, logit)
    return weights[:, None]
